# Initial kernel scaffold; baseline (speedup 1.0000x reference)
import jax
import jax.numpy as jnp
from jax.experimental import pallas as pl


def kernel(node_attr, edge_index, edge_attr, edge_sh, W1, b1, W2, b2):
    raise NotImplementedError("write your pallas kernel here")


# speedup vs baseline: 1.5611x; 1.5611x over previous
import functools

import jax
import jax.numpy as jnp
from jax import lax
from jax.experimental import pallas as pl
from jax.experimental.pallas import tpu as pltpu
from jax.experimental.pallas import tpu_sc as plsc

D = 16
NC = 2
NS = 16
NW = NC * NS
CHUNK = 128
ALPHA = 1.0 / (D ** 0.5)


def _mesh():
    return plsc.VectorSubcoreMesh(
        core_axis_name="c", subcore_axis_name="s", num_cores=NC,
        num_subcores=NS)


def _make_gather(ep, n_nodes):
    e_per_w = ep // NW
    ch_per_w = e_per_w // CHUNK

    @functools.partial(
        pl.kernel, mesh=_mesh(),
        compiler_params=pltpu.CompilerParams(use_tc_tiling_on_sc=False),
        out_type=jax.ShapeDtypeStruct((ep, D), jnp.float32),
        scratch_types=[
            pltpu.VMEM((ch_per_w, CHUNK), jnp.int32),
            pltpu.VMEM((e_per_w, D), jnp.float32),
            pltpu.SemaphoreType.DMA,
        ],
    )
    def gather_kernel(idx_hbm, table_hbm, x_hbm, idx_v, rows_v, sem):
        wid = lax.axis_index("s") * NC + lax.axis_index("c")
        pltpu.sync_copy(idx_hbm.at[pl.ds(wid * ch_per_w, ch_per_w)], idx_v)

        def body(j, carry):
            pltpu.async_copy(
                table_hbm.at[idx_v.at[j]],
                rows_v.at[pl.ds(j * CHUNK, CHUNK)], sem).wait()
            return carry

        lax.fori_loop(0, ch_per_w, body, 0)
        pltpu.sync_copy(rows_v, x_hbm.at[pl.ds(wid * e_per_w, e_per_w)])

    return gather_kernel


def _make_scatter(ep, n_pad):
    e_per_w = ep // NW
    ch_per_w = e_per_w // CHUNK
    zrows = n_pad // NS

    @functools.partial(
        pl.kernel, mesh=_mesh(),
        compiler_params=pltpu.CompilerParams(use_tc_tiling_on_sc=False),
        out_type=(jax.ShapeDtypeStruct((NC, n_pad, D), jnp.float32),
                  jax.ShapeDtypeStruct((NC, n_pad), jnp.float32)),
        scratch_types=[
            pltpu.VMEM((ch_per_w, CHUNK), jnp.int32),
            pltpu.VMEM((e_per_w, D), jnp.float32),
            pltpu.VMEM((zrows, D), jnp.float32),
            pltpu.VMEM((zrows,), jnp.float32),
            pltpu.VMEM((CHUNK,), jnp.float32),
            pltpu.VMEM_SHARED((n_pad, D), jnp.float32),
            pltpu.VMEM_SHARED((n_pad,), jnp.float32),
        ],
    )
    def scatter_kernel(idx_hbm, tp_hbm, psum_hbm, pcnt_hbm,
                       idx_v, rows_v, zbuf, zbufc, ones_v, acc_sh, cnt_sh):
        cid = lax.axis_index("c")
        sid = lax.axis_index("s")
        wid = sid * NC + cid
        zero16 = jnp.zeros((16,), jnp.float32)
        one16 = jnp.ones((16,), jnp.float32)

        def zrow(i, carry):
            zbuf[i, :] = zero16
            return carry

        lax.fori_loop(0, zrows, zrow, 0)

        def zcnt(i, carry):
            zbufc[pl.ds(i * 16, 16)] = zero16
            return carry

        lax.fori_loop(0, zrows // 16, zcnt, 0)

        def orow(i, carry):
            ones_v[pl.ds(i * 16, 16)] = one16
            return carry

        lax.fori_loop(0, CHUNK // 16, orow, 0)

        pltpu.sync_copy(zbuf, acc_sh.at[pl.ds(sid * zrows, zrows)])
        pltpu.sync_copy(zbufc, cnt_sh.at[pl.ds(sid * zrows, zrows)])
        plsc.subcore_barrier()

        pltpu.sync_copy(idx_hbm.at[pl.ds(wid * ch_per_w, ch_per_w)], idx_v)
        pltpu.sync_copy(tp_hbm.at[pl.ds(wid * e_per_w, e_per_w)], rows_v)

        def body(j, carry):
            pltpu.sync_copy(rows_v.at[pl.ds(j * CHUNK, CHUNK)],
                            acc_sh.at[idx_v.at[j]], add=True)
            pltpu.sync_copy(ones_v, cnt_sh.at[idx_v.at[j]], add=True)
            return carry

        lax.fori_loop(0, ch_per_w, body, 0)
        plsc.subcore_barrier()

        pltpu.sync_copy(acc_sh.at[pl.ds(sid * zrows, zrows)],
                        psum_hbm.at[cid].at[pl.ds(sid * zrows, zrows)])
        pltpu.sync_copy(cnt_sh.at[pl.ds(sid * zrows, zrows)],
                        pcnt_hbm.at[cid].at[pl.ds(sid * zrows, zrows)])

    return scatter_kernel


def _tp_body(ea_ref, x_ref, sh_ref, w1_ref, b1_ref, w2_ref, b2_ref, tp_ref):
    h = jnp.maximum(
        jnp.dot(ea_ref[...], w1_ref[...],
                preferred_element_type=jnp.float32) + b1_ref[...], 0.0)
    tpw = jnp.dot(h, w2_ref[...],
                  preferred_element_type=jnp.float32) + b2_ref[...]
    xs = x_ref[...] * sh_ref[...] * ALPHA
    acc = xs[:, 0:1] * tpw[:, 0:D]
    for i in range(1, D):
        acc = acc + xs[:, i:i + 1] * tpw[:, i * D:(i + 1) * D]
    tp_ref[...] = acc


def _run_tp(ea, x, sh, w1, b1, w2, b2, blk=2048):
    ep, f = ea.shape
    wn = w2.shape[1]
    grid = (ep // blk,)
    return pl.pallas_call(
        _tp_body,
        grid=grid,
        in_specs=[
            pl.BlockSpec((blk, f), lambda i: (i, 0)),
            pl.BlockSpec((blk, D), lambda i: (i, 0)),
            pl.BlockSpec((blk, 1), lambda i: (i, 0)),
            pl.BlockSpec((f, f), lambda i: (0, 0)),
            pl.BlockSpec((1, f), lambda i: (0, 0)),
            pl.BlockSpec((f, wn), lambda i: (0, 0)),
            pl.BlockSpec((1, wn), lambda i: (0, 0)),
        ],
        out_specs=pl.BlockSpec((blk, D), lambda i: (i, 0)),
        out_shape=jax.ShapeDtypeStruct((ep, D), jnp.float32),
    )(ea, x, sh, w1, b1, w2, b2)


def _combine_body(ps_ref, pc_ref, na_ref, out_ref):
    p = ps_ref[...]
    c = pc_ref[...]
    cnt = jnp.maximum(c[0] + c[1], 1.0)
    out_ref[...] = (p[0] + p[1]) / cnt[:, None] + na_ref[...]


def _run_combine(psum, pcnt, na_pad, rb=1024):
    n_pad = psum.shape[1]
    grid = (n_pad // rb,)
    return pl.pallas_call(
        _combine_body,
        grid=grid,
        in_specs=[
            pl.BlockSpec((NC, rb, D), lambda i: (0, i, 0)),
            pl.BlockSpec((NC, rb), lambda i: (0, i)),
            pl.BlockSpec((rb, D), lambda i: (i, 0)),
        ],
        out_specs=pl.BlockSpec((rb, D), lambda i: (i, 0)),
        out_shape=jax.ShapeDtypeStruct((n_pad, D), jnp.float32),
    )(psum, pcnt, na_pad)


def kernel(node_attr, edge_index, edge_attr, edge_sh, W1, b1, W2, b2):
    n_nodes, d = node_attr.shape
    e = edge_attr.shape[0]
    ep = ((e + NW * CHUNK - 1) // (NW * CHUNK)) * (NW * CHUNK)
    n_pad = ((n_nodes + 1 + NW * 16 - 1) // (NW * 16)) * (NW * 16)
    pad = ep - e

    src = edge_index[0].astype(jnp.int32)
    dst = edge_index[1].astype(jnp.int32)
    dst_p = jnp.concatenate(
        [dst, jnp.zeros((pad,), jnp.int32)]).reshape(ep // CHUNK, CHUNK)
    src_p = jnp.concatenate(
        [src, jnp.full((pad,), n_nodes, jnp.int32)]).reshape(
            ep // CHUNK, CHUNK)
    ea_p = jnp.concatenate(
        [edge_attr, jnp.zeros((pad, edge_attr.shape[1]), jnp.float32)])
    sh_p = jnp.concatenate(
        [edge_sh, jnp.zeros((pad, edge_sh.shape[1]), jnp.float32)])
    na_pad = jnp.concatenate(
        [node_attr, jnp.zeros((n_pad - n_nodes, d), jnp.float32)])

    x = _make_gather(ep, n_nodes)(dst_p, node_attr)
    tp = _run_tp(ea_p, x, sh_p, W1, b1.reshape(1, -1), W2, b2.reshape(1, -1))
    psum, pcnt = _make_scatter(ep, n_pad)(src_p, tp)
    out_pad = _run_combine(psum, pcnt, na_pad)
    return out_pad[:n_nodes]


# speedup vs baseline: 3.4913x; 2.2365x over previous
import functools

import jax
import jax.numpy as jnp
from jax import lax
from jax.experimental import pallas as pl
from jax.experimental.pallas import tpu as pltpu
from jax.experimental.pallas import tpu_sc as plsc

D = 16
NC = 2
NS = 16
NW = NC * NS
CHUNK = 128
ALPHA = 1.0 / (D ** 0.5)


def _mesh():
    return plsc.VectorSubcoreMesh(
        core_axis_name="c", subcore_axis_name="s", num_cores=NC,
        num_subcores=NS)


def _make_gather(ep, n_nodes):
    e_per_w = ep // NW
    ch_per_w = e_per_w // CHUNK

    @functools.partial(
        pl.kernel, mesh=_mesh(),
        compiler_params=pltpu.CompilerParams(use_tc_tiling_on_sc=False),
        out_type=jax.ShapeDtypeStruct((ep, D), jnp.float32),
        scratch_types=[
            pltpu.VMEM((ch_per_w, CHUNK), jnp.int32),
            pltpu.VMEM((e_per_w, D), jnp.float32),
            pltpu.SemaphoreType.DMA,
        ],
    )
    def gather_kernel(idx_hbm, table_hbm, x_hbm, idx_v, rows_v, sem):
        wid = lax.axis_index("s") * NC + lax.axis_index("c")
        pltpu.sync_copy(idx_hbm.at[pl.ds(wid * ch_per_w, ch_per_w)], idx_v)

        def fire(j, carry):
            pltpu.async_copy(
                table_hbm.at[idx_v.at[j]],
                rows_v.at[pl.ds(j * CHUNK, CHUNK)], sem)
            return carry

        lax.fori_loop(0, ch_per_w, fire, 0)
        pltpu.make_async_copy(
            x_hbm.at[pl.ds(wid * e_per_w, e_per_w)], rows_v, sem).wait()
        pltpu.sync_copy(rows_v, x_hbm.at[pl.ds(wid * e_per_w, e_per_w)])

    return gather_kernel


def _make_scatter(ep, n_pad):
    e_per_w = ep // NW
    ch_per_w = e_per_w // CHUNK
    zrows = n_pad // NS

    @functools.partial(
        pl.kernel, mesh=_mesh(),
        compiler_params=pltpu.CompilerParams(use_tc_tiling_on_sc=False),
        out_type=(jax.ShapeDtypeStruct((NC, n_pad, D), jnp.float32),
                  jax.ShapeDtypeStruct((NC, n_pad), jnp.float32)),
        scratch_types=[
            pltpu.VMEM((ch_per_w, CHUNK), jnp.int32),
            pltpu.VMEM((e_per_w, D), jnp.float32),
            pltpu.VMEM((zrows, D), jnp.float32),
            pltpu.VMEM((zrows,), jnp.float32),
            pltpu.VMEM((CHUNK,), jnp.float32),
            pltpu.VMEM_SHARED((n_pad, D), jnp.float32),
            pltpu.VMEM_SHARED((n_pad,), jnp.float32),
        ],
    )
    def scatter_kernel(idx_hbm, tp_hbm, psum_hbm, pcnt_hbm,
                       idx_v, rows_v, zbuf, zbufc, ones_v, acc_sh, cnt_sh):
        cid = lax.axis_index("c")
        sid = lax.axis_index("s")
        wid = sid * NC + cid
        zero16 = jnp.zeros((16,), jnp.float32)
        one16 = jnp.ones((16,), jnp.float32)

        def zrow(i, carry):
            zbuf[i, :] = zero16
            return carry

        lax.fori_loop(0, zrows, zrow, 0)

        def zcnt(i, carry):
            zbufc[pl.ds(i * 16, 16)] = zero16
            return carry

        lax.fori_loop(0, zrows // 16, zcnt, 0)

        def orow(i, carry):
            ones_v[pl.ds(i * 16, 16)] = one16
            return carry

        lax.fori_loop(0, CHUNK // 16, orow, 0)

        pltpu.sync_copy(zbuf, acc_sh.at[pl.ds(sid * zrows, zrows)])
        pltpu.sync_copy(zbufc, cnt_sh.at[pl.ds(sid * zrows, zrows)])
        plsc.subcore_barrier()

        pltpu.sync_copy(idx_hbm.at[pl.ds(wid * ch_per_w, ch_per_w)], idx_v)
        pltpu.sync_copy(tp_hbm.at[pl.ds(wid * e_per_w, e_per_w)], rows_v)

        def body(j, carry):
            pltpu.sync_copy(rows_v.at[pl.ds(j * CHUNK, CHUNK)],
                            acc_sh.at[idx_v.at[j]], add=True)
            pltpu.sync_copy(ones_v, cnt_sh.at[idx_v.at[j]], add=True)
            return carry

        lax.fori_loop(0, ch_per_w, body, 0)
        plsc.subcore_barrier()

        pltpu.sync_copy(acc_sh.at[pl.ds(sid * zrows, zrows)],
                        psum_hbm.at[cid].at[pl.ds(sid * zrows, zrows)])
        pltpu.sync_copy(cnt_sh.at[pl.ds(sid * zrows, zrows)],
                        pcnt_hbm.at[cid].at[pl.ds(sid * zrows, zrows)])

    return scatter_kernel


def _tp_body(ea_ref, x_ref, sh_ref, w1t_ref, b1t_ref, w2t_ref, b2t_ref,
             tp_ref):
    eaT = jnp.transpose(ea_ref[...])
    xsT = jnp.transpose(x_ref[...] * sh_ref[...]) * ALPHA
    hT = jnp.maximum(
        jnp.dot(w1t_ref[...], eaT,
                preferred_element_type=jnp.float32) + b1t_ref[...], 0.0)
    tpwT = jnp.dot(w2t_ref[...], hT.astype(jnp.bfloat16),
                   preferred_element_type=jnp.float32) + b2t_ref[...]
    acc = xsT[0:1, :] * tpwT[0:D, :]
    for i in range(1, D):
        acc = acc + xsT[i:i + 1, :] * tpwT[i * D:(i + 1) * D, :]
    tp_ref[...] = jnp.transpose(acc)


def _run_tp(ea, x, sh, w1t, b1t, w2t, b2t, blk=2048):
    ep, f = ea.shape
    wn = w2t.shape[0]
    grid = (ep // blk,)
    return pl.pallas_call(
        _tp_body,
        grid=grid,
        in_specs=[
            pl.BlockSpec((blk, f), lambda i: (i, 0)),
            pl.BlockSpec((blk, D), lambda i: (i, 0)),
            pl.BlockSpec((blk, 1), lambda i: (i, 0)),
            pl.BlockSpec((f, f), lambda i: (0, 0)),
            pl.BlockSpec((f, 1), lambda i: (0, 0)),
            pl.BlockSpec((wn, f), lambda i: (0, 0)),
            pl.BlockSpec((wn, 1), lambda i: (0, 0)),
        ],
        out_specs=pl.BlockSpec((blk, D), lambda i: (i, 0)),
        out_shape=jax.ShapeDtypeStruct((ep, D), jnp.float32),
    )(ea, x, sh, w1t, b1t, w2t, b2t)


def _combine_body(ps_ref, pc_ref, na_ref, out_ref):
    p = ps_ref[...]
    c = pc_ref[...]
    cnt = jnp.maximum(c[0] + c[1], 1.0)
    out_ref[...] = (p[0] + p[1]) / cnt[:, None] + na_ref[...]


def _run_combine(psum, pcnt, na_pad, rb=1024):
    n_pad = psum.shape[1]
    grid = (n_pad // rb,)
    return pl.pallas_call(
        _combine_body,
        grid=grid,
        in_specs=[
            pl.BlockSpec((NC, rb, D), lambda i: (0, i, 0)),
            pl.BlockSpec((NC, rb), lambda i: (0, i)),
            pl.BlockSpec((rb, D), lambda i: (i, 0)),
        ],
        out_specs=pl.BlockSpec((rb, D), lambda i: (i, 0)),
        out_shape=jax.ShapeDtypeStruct((n_pad, D), jnp.float32),
    )(psum, pcnt, na_pad)


def kernel(node_attr, edge_index, edge_attr, edge_sh, W1, b1, W2, b2):
    n_nodes, d = node_attr.shape
    e = edge_attr.shape[0]
    ep = ((e + NW * CHUNK - 1) // (NW * CHUNK)) * (NW * CHUNK)
    n_pad = ((n_nodes + 1 + NW * 16 - 1) // (NW * 16)) * (NW * 16)
    pad = ep - e

    src = edge_index[0].astype(jnp.int32)
    dst = edge_index[1].astype(jnp.int32)
    dst_p = jnp.concatenate(
        [dst, jnp.zeros((pad,), jnp.int32)]).reshape(ep // CHUNK, CHUNK)
    src_p = jnp.concatenate(
        [src, jnp.full((pad,), n_nodes, jnp.int32)]).reshape(
            ep // CHUNK, CHUNK)
    ea_p = jnp.concatenate(
        [edge_attr, jnp.zeros((pad, edge_attr.shape[1]), jnp.float32)])
    sh_p = jnp.concatenate(
        [edge_sh, jnp.zeros((pad, edge_sh.shape[1]), jnp.float32)])
    na_pad = jnp.concatenate(
        [node_attr, jnp.zeros((n_pad - n_nodes, d), jnp.float32)])

    x = _make_gather(ep, n_nodes)(dst_p, node_attr)
    tp = _run_tp(ea_p, x, sh_p, W1.T, b1[:, None],
                 W2.T.astype(jnp.bfloat16), b2[:, None])
    psum, pcnt = _make_scatter(ep, n_pad)(src_p, tp)
    out_pad = _run_combine(psum, pcnt, na_pad)
    return out_pad[:n_nodes]


# speedup vs baseline: 5.9688x; 1.7096x over previous
import functools

import jax
import jax.numpy as jnp
from jax import lax
from jax.experimental import pallas as pl
from jax.experimental.pallas import tpu as pltpu
from jax.experimental.pallas import tpu_sc as plsc

D = 16
NC = 2
NS = 16
NW = NC * NS
CHUNK = 125
ALPHA = 1.0 / (D ** 0.5)


def _mesh():
    return plsc.VectorSubcoreMesh(
        core_axis_name="c", subcore_axis_name="s", num_cores=NC,
        num_subcores=NS)


def _make_gather(ep, n_nodes):
    e_per_w = ep // NW
    ch_per_w = e_per_w // CHUNK

    @functools.partial(
        pl.kernel, mesh=_mesh(),
        compiler_params=pltpu.CompilerParams(use_tc_tiling_on_sc=False),
        out_type=jax.ShapeDtypeStruct((ep, D), jnp.float32),
        scratch_types=[
            pltpu.VMEM((ch_per_w, CHUNK), jnp.int32),
            pltpu.VMEM((e_per_w, D), jnp.float32),
            pltpu.SemaphoreType.DMA,
        ],
    )
    def gather_kernel(idx_hbm, table_hbm, x_hbm, idx_v, rows_v, sem):
        wid = lax.axis_index("s") * NC + lax.axis_index("c")
        pltpu.sync_copy(idx_hbm.at[pl.ds(wid * ch_per_w, ch_per_w)], idx_v)

        def fire(j, carry):
            pltpu.async_copy(
                table_hbm.at[idx_v.at[j]],
                rows_v.at[pl.ds(j * CHUNK, CHUNK)], sem)
            return carry

        lax.fori_loop(0, ch_per_w, fire, 0)
        pltpu.make_async_copy(
            x_hbm.at[pl.ds(wid * e_per_w, e_per_w)], rows_v, sem).wait()
        pltpu.sync_copy(rows_v, x_hbm.at[pl.ds(wid * e_per_w, e_per_w)])

    return gather_kernel


def _make_scatter(ep, n_pad):
    e_per_w = ep // NW
    ch_per_w = e_per_w // CHUNK
    zrows = n_pad // NS

    @functools.partial(
        pl.kernel, mesh=_mesh(),
        compiler_params=pltpu.CompilerParams(use_tc_tiling_on_sc=False),
        out_type=(jax.ShapeDtypeStruct((NC, n_pad, D), jnp.float32),
                  jax.ShapeDtypeStruct((NC, n_pad), jnp.float32)),
        scratch_types=[
            pltpu.VMEM((ch_per_w, CHUNK), jnp.int32),
            pltpu.VMEM((e_per_w, D), jnp.float32),
            pltpu.VMEM((zrows, D), jnp.float32),
            pltpu.VMEM((zrows,), jnp.float32),
            pltpu.VMEM((((CHUNK + 15) // 16) * 16,), jnp.float32),
            pltpu.VMEM_SHARED((n_pad, D), jnp.float32),
            pltpu.VMEM_SHARED((n_pad,), jnp.float32),
        ],
    )
    def scatter_kernel(idx_hbm, tp_hbm, psum_hbm, pcnt_hbm,
                       idx_v, rows_v, zbuf, zbufc, ones_v, acc_sh, cnt_sh):
        cid = lax.axis_index("c")
        sid = lax.axis_index("s")
        wid = sid * NC + cid
        zero16 = jnp.zeros((16,), jnp.float32)
        one16 = jnp.ones((16,), jnp.float32)

        def zrow(i, carry):
            zbuf[i, :] = zero16
            return carry

        lax.fori_loop(0, zrows, zrow, 0)

        def zcnt(i, carry):
            zbufc[pl.ds(i * 16, 16)] = zero16
            return carry

        lax.fori_loop(0, zrows // 16, zcnt, 0)

        def orow(i, carry):
            ones_v[pl.ds(i * 16, 16)] = one16
            return carry

        lax.fori_loop(0, (CHUNK + 15) // 16, orow, 0)

        pltpu.sync_copy(zbuf, acc_sh.at[pl.ds(sid * zrows, zrows)])
        pltpu.sync_copy(zbufc, cnt_sh.at[pl.ds(sid * zrows, zrows)])
        plsc.subcore_barrier()

        pltpu.sync_copy(idx_hbm.at[pl.ds(wid * ch_per_w, ch_per_w)], idx_v)
        pltpu.sync_copy(tp_hbm.at[pl.ds(wid * e_per_w, e_per_w)], rows_v)

        def body(j, carry):
            pltpu.sync_copy(rows_v.at[pl.ds(j * CHUNK, CHUNK)],
                            acc_sh.at[idx_v.at[j]], add=True)
            pltpu.sync_copy(ones_v.at[pl.ds(0, CHUNK)],
                            cnt_sh.at[idx_v.at[j]], add=True)
            return carry

        lax.fori_loop(0, ch_per_w, body, 0)
        plsc.subcore_barrier()

        pltpu.sync_copy(acc_sh.at[pl.ds(sid * zrows, zrows)],
                        psum_hbm.at[cid].at[pl.ds(sid * zrows, zrows)])
        pltpu.sync_copy(cnt_sh.at[pl.ds(sid * zrows, zrows)],
                        pcnt_hbm.at[cid].at[pl.ds(sid * zrows, zrows)])

    return scatter_kernel


def _tp_body(ea_ref, x_ref, sh_ref, w1t_ref, b1t_ref, w2t_ref, b2t_ref,
             tp_ref):
    eaT = ea_ref[...]
    xsT = jnp.transpose(x_ref[...]) * sh_ref[...] * ALPHA
    hT = jnp.maximum(
        jnp.dot(w1t_ref[...], eaT,
                preferred_element_type=jnp.float32) + b1t_ref[...], 0.0)
    tpwT = jnp.dot(w2t_ref[...], hT.astype(jnp.bfloat16),
                   preferred_element_type=jnp.float32) + b2t_ref[...]
    acc = xsT[0:1, :] * tpwT[0:D, :]
    for i in range(1, D):
        acc = acc + xsT[i:i + 1, :] * tpwT[i * D:(i + 1) * D, :]
    tp_ref[...] = jnp.transpose(acc)


def _run_tp(eaT, x, shT, w1t, b1t, w2t, b2t, off=0, blk=3200):
    f, ep = eaT.shape
    e_half = x.shape[0]
    wn = w2t.shape[0]
    grid = (e_half // blk,)
    return pl.pallas_call(
        _tp_body,
        grid=grid,
        in_specs=[
            pl.BlockSpec((f, blk), lambda i: (0, i + off)),
            pl.BlockSpec((blk, D), lambda i: (i, 0)),
            pl.BlockSpec((1, blk), lambda i: (0, i + off)),
            pl.BlockSpec((f, f), lambda i: (0, 0)),
            pl.BlockSpec((f, 1), lambda i: (0, 0)),
            pl.BlockSpec((wn, f), lambda i: (0, 0)),
            pl.BlockSpec((wn, 1), lambda i: (0, 0)),
        ],
        out_specs=pl.BlockSpec((blk, D), lambda i: (i, 0)),
        out_shape=jax.ShapeDtypeStruct((e_half, D), jnp.float32),
    )(eaT, x, shT, w1t, b1t, w2t, b2t)


def _make_combine(n_pad):
    rows_w = n_pad // NW

    @functools.partial(
        pl.kernel, mesh=_mesh(),
        compiler_params=pltpu.CompilerParams(
            use_tc_tiling_on_sc=False, needs_layout_passes=False),
        out_type=jax.ShapeDtypeStruct((n_pad, D), jnp.float32),
        scratch_types=[
            pltpu.VMEM((rows_w, D), jnp.float32),
            pltpu.VMEM((rows_w, D), jnp.float32),
            pltpu.VMEM((rows_w,), jnp.float32),
            pltpu.VMEM((rows_w,), jnp.float32),
            pltpu.VMEM((rows_w,), jnp.float32),
            pltpu.VMEM((rows_w, D), jnp.float32),
            pltpu.VMEM((rows_w, D), jnp.float32),
        ],
    )
    def combine_kernel(psa_hbm, pca_hbm, psb_hbm, pcb_hbm, na_hbm, out_hbm,
                       p0v, p1v, c0v, c1v, invv, nav, outv):
        wid = lax.axis_index("s") * NC + lax.axis_index("c")
        base = wid * rows_w
        csum = invv

        pltpu.sync_copy(pca_hbm.at[0].at[pl.ds(base, rows_w)], c0v)
        pltpu.sync_copy(pca_hbm.at[1].at[pl.ds(base, rows_w)], c1v)

        def grp0(g, carry):
            csum[pl.ds(g * 16, 16)] = (c0v[pl.ds(g * 16, 16)]
                                       + c1v[pl.ds(g * 16, 16)])
            return carry

        lax.fori_loop(0, rows_w // 16, grp0, 0)
        pltpu.sync_copy(pcb_hbm.at[0].at[pl.ds(base, rows_w)], c0v)
        pltpu.sync_copy(pcb_hbm.at[1].at[pl.ds(base, rows_w)], c1v)

        def grp1(g, carry):
            c = (csum[pl.ds(g * 16, 16)] + c0v[pl.ds(g * 16, 16)]
                 + c1v[pl.ds(g * 16, 16)])
            csum[pl.ds(g * 16, 16)] = 1.0 / jnp.maximum(c, 1.0)
            return carry

        lax.fori_loop(0, rows_w // 16, grp1, 0)

        pltpu.sync_copy(psa_hbm.at[0].at[pl.ds(base, rows_w)], p0v)
        pltpu.sync_copy(psa_hbm.at[1].at[pl.ds(base, rows_w)], p1v)
        pltpu.sync_copy(na_hbm.at[pl.ds(base, rows_w)], nav)

        def rowa(r, carry):
            iv = plsc.load_gather(invv, [jnp.full((16,), r, jnp.int32)])
            outv[r, :] = (p0v[r, :] + p1v[r, :]) * iv + nav[r, :]
            return carry

        lax.fori_loop(0, rows_w, rowa, 0)
        pltpu.sync_copy(psb_hbm.at[0].at[pl.ds(base, rows_w)], p0v)
        pltpu.sync_copy(psb_hbm.at[1].at[pl.ds(base, rows_w)], p1v)

        def rowb(r, carry):
            iv = plsc.load_gather(invv, [jnp.full((16,), r, jnp.int32)])
            outv[r, :] = outv[r, :] + (p0v[r, :] + p1v[r, :]) * iv
            return carry

        lax.fori_loop(0, rows_w, rowb, 0)
        pltpu.sync_copy(outv, out_hbm.at[pl.ds(base, rows_w)])

    return combine_kernel


def kernel(node_attr, edge_index, edge_attr, edge_sh, W1, b1, W2, b2):
    n_nodes, d = node_attr.shape
    e = edge_attr.shape[0]
    eh = e // 2
    n_pad = ((n_nodes + 1 + NW * D - 1) // (NW * D)) * (NW * D)

    src = edge_index[0].astype(jnp.int32)
    dst = edge_index[1].astype(jnp.int32)
    na_pad = jnp.concatenate(
        [node_attr, jnp.zeros((n_pad - n_nodes, d), jnp.float32)])
    eaT = edge_attr.T
    shT = edge_sh.T
    w1t, b1t = W1.T, b1[:, None]
    w2t, b2t = W2.T.astype(jnp.bfloat16), b2[:, None]

    halves = []
    for h in range(2):
        dsth = lax.slice(dst, (h * eh,), ((h + 1) * eh,)).reshape(
            eh // CHUNK, CHUNK)
        srch = lax.slice(src, (h * eh,), ((h + 1) * eh,)).reshape(
            eh // CHUNK, CHUNK)
        xh = _make_gather(eh, n_nodes)(dsth, node_attr)
        tph = _run_tp(eaT, xh, shT, w1t, b1t, w2t, b2t,
                      off=h * (eh // 3200))
        halves.append(_make_scatter(eh, n_pad)(srch, tph))

    (psa, pca), (psb, pcb) = halves
    out_pad = _make_combine(n_pad)(psa, pca, psb, pcb, na_pad)
    return out_pad[:n_nodes]
